# Initial kernel scaffold; baseline (speedup 1.0000x reference)
import jax
import jax.numpy as jnp
from jax.experimental import pallas as pl


def kernel(in_features, embedding_property, emb_table, W_init, W_f1, b_f1, W_f2, b_f2, W_o1, b_o1, W_o2, b_o2):
    raise NotImplementedError("write your pallas kernel here")


# speedup vs baseline: 1.6940x; 1.6940x over previous
import functools

import jax
import jax.numpy as jnp
import numpy as np
from jax.experimental import pallas as pl

_N_GAUSS = 50
_CUTOFF = 5.0
_VARIANCE = 1.0
_LOG2 = float(np.log(2.0))


def _ssp(x):
    return jnp.maximum(x, 0.0) + jnp.log1p(jnp.exp(-jnp.abs(x))) - _LOG2


def _schnet_body(xi_ref, xj_ref, ep_ref, emb_ref, winit_ref, wf1_ref,
                 wf2_ref, bias_ref, wo1_ref, wo2_ref, out_ref, *, B, FEAT):
    E = B * B
    G = 64

    xi = xi_ref[0]
    xj = xj_ref[0]
    diff = xi - xj
    d2 = jnp.sum(diff * diff, axis=1, keepdims=True)
    d = jnp.sqrt(d2)

    e_ids = jax.lax.broadcasted_iota(jnp.int32, (E, 1), 0)
    i_ids = e_ids // B
    j_ids = e_ids - i_ids * B
    mask = jnp.where((d < _CUTOFF) & (i_ids != j_ids), 1.0, 0.0)

    g = jax.lax.broadcasted_iota(jnp.int32, (1, G), 1).astype(jnp.float32)
    centers = g * (_CUTOFF / (_N_GAUSS - 1))
    rbf = jnp.exp((-0.5 / _VARIANCE) * (d - centers) ** 2)

    z1 = jnp.dot(rbf, wf1_ref[...], preferred_element_type=jnp.float32)
    z1 = z1 + bias_ref[0:1, :]
    a1 = _ssp(z1)
    filt = jnp.dot(a1, wf2_ref[...], preferred_element_type=jnp.float32)
    filt = filt + bias_ref[1:2, :]
    filt = filt * mask

    ep = ep_ref[0]
    vocab = jax.lax.broadcasted_iota(jnp.int32, (B, 64), 1)
    onehot = jnp.where(ep == vocab, 1.0, 0.0)
    feat = jnp.dot(onehot, emb_ref[...], preferred_element_type=jnp.float32)

    for b in range(2):
        h = jnp.dot(feat, winit_ref[b], preferred_element_type=jnp.float32)
        fb = filt[:, b * FEAT:(b + 1) * FEAT].reshape(B, B, FEAT)
        conv = fb * h[None, :, :]
        agg = jnp.sum(conv, axis=1)
        t = jnp.dot(agg, wo1_ref[b], preferred_element_type=jnp.float32)
        t = _ssp(t + bias_ref[2 + 2 * b:3 + 2 * b, :FEAT])
        out = jnp.dot(t, wo2_ref[b], preferred_element_type=jnp.float32)
        out = out + bias_ref[3 + 2 * b:4 + 2 * b, :FEAT]
        feat = feat + out

    out_ref[0] = feat


def kernel(in_features, embedding_property, emb_table, W_init, W_f1, b_f1,
           W_f2, b_f2, W_o1, b_o1, W_o2, b_o2):
    Fr, B, _ = in_features.shape
    N_EMB, FEAT = emb_table.shape
    E = B * B
    G = 64

    x = in_features
    XI = jnp.broadcast_to(x[:, :, None, :], (Fr, B, B, 3)).reshape(Fr, E, 3)
    XJ = jnp.broadcast_to(x[:, None, :, :], (Fr, B, B, 3)).reshape(Fr, E, 3)
    ep3 = embedding_property.astype(jnp.int32).reshape(Fr, B, 1)
    emb_pad = jnp.pad(emb_table, ((0, 64 - N_EMB), (0, 0)))

    wf1p = jnp.pad(W_f1, ((0, 0), (0, G - _N_GAUSS), (0, 0)))
    W_f1c = jnp.concatenate([wf1p[0], wf1p[1]], axis=1)
    W_f2c = jnp.zeros((2 * FEAT, 2 * FEAT), jnp.float32)
    W_f2c = W_f2c.at[:FEAT, :FEAT].set(W_f2[0]).at[FEAT:, FEAT:].set(W_f2[1])

    def pad128(v):
        return jnp.pad(v, (0, 2 * FEAT - v.shape[0]))

    bias_pack = jnp.stack([
        jnp.concatenate([b_f1[0], b_f1[1]]),
        jnp.concatenate([b_f2[0], b_f2[1]]),
        pad128(b_o1[0]), pad128(b_o2[0]),
        pad128(b_o1[1]), pad128(b_o2[1]),
        jnp.zeros(2 * FEAT), jnp.zeros(2 * FEAT),
    ])

    body = functools.partial(_schnet_body, B=B, FEAT=FEAT)
    out = pl.pallas_call(
        body,
        grid=(Fr,),
        in_specs=[
            pl.BlockSpec((1, E, 3), lambda f: (f, 0, 0)),
            pl.BlockSpec((1, E, 3), lambda f: (f, 0, 0)),
            pl.BlockSpec((1, B, 1), lambda f: (f, 0, 0)),
            pl.BlockSpec((64, FEAT), lambda f: (0, 0)),
            pl.BlockSpec((2, FEAT, FEAT), lambda f: (0, 0, 0)),
            pl.BlockSpec((G, 2 * FEAT), lambda f: (0, 0)),
            pl.BlockSpec((2 * FEAT, 2 * FEAT), lambda f: (0, 0)),
            pl.BlockSpec((8, 2 * FEAT), lambda f: (0, 0)),
            pl.BlockSpec((2, FEAT, FEAT), lambda f: (0, 0, 0)),
            pl.BlockSpec((2, FEAT, FEAT), lambda f: (0, 0, 0)),
        ],
        out_specs=pl.BlockSpec((1, B, FEAT), lambda f: (f, 0, 0)),
        out_shape=jax.ShapeDtypeStruct((Fr, B, FEAT), jnp.float32),
    )(XI, XJ, ep3, emb_pad, W_init, W_f1c, W_f2c, bias_pack, W_o1, W_o2)
    return out


# speedup vs baseline: 1.9461x; 1.1488x over previous
import functools

import jax
import jax.numpy as jnp
import numpy as np
from jax.experimental import pallas as pl

_N_GAUSS = 50
_CUTOFF = 5.0
_VARIANCE = 1.0
_LOG2 = float(np.log(2.0))


def _ssp(x):
    return jnp.maximum(x, 0.0) + jnp.log1p(jnp.exp(-jnp.abs(x))) - _LOG2


def _schnet_body(xi_ref, xj_ref, ep_ref, emb_ref, winit_ref, wf1_ref,
                 wf2_ref, bias_ref, wo1_ref, wo2_ref, m_ref, s_ref, out_ref,
                 *, P, B, FEAT):
    G = 64

    xi = xi_ref[0]
    xj = xj_ref[0]
    diff = xi - xj
    d2 = jnp.sum(diff * diff, axis=1, keepdims=True)
    d = jnp.sqrt(d2)
    mask = jnp.where(d < _CUTOFF, 1.0, 0.0)

    g = jax.lax.broadcasted_iota(jnp.int32, (1, G), 1).astype(jnp.float32)
    centers = g * (_CUTOFF / (_N_GAUSS - 1))
    rbf = jnp.exp((-0.5 / _VARIANCE) * (d - centers) ** 2)

    z1 = jnp.dot(rbf, wf1_ref[...], preferred_element_type=jnp.float32)
    z1 = z1 + bias_ref[0:1, :]
    a1 = _ssp(z1)
    filt = jnp.dot(a1, wf2_ref[...], preferred_element_type=jnp.float32)
    filt = filt + bias_ref[1:2, :]
    filt = filt * mask

    ep = ep_ref[0]
    vocab = jax.lax.broadcasted_iota(jnp.int32, (B, 64), 1)
    onehot = jnp.where(ep == vocab, 1.0, 0.0)
    feat = jnp.dot(onehot, emb_ref[...], preferred_element_type=jnp.float32)

    M = m_ref[...]
    S = s_ref[...]

    for b in range(2):
        h = jnp.dot(feat, winit_ref[b], preferred_element_type=jnp.float32)
        fb = filt[:, b * FEAT:(b + 1) * FEAT]
        hsum = jnp.dot(S, h, preferred_element_type=jnp.float32)
        kf = jnp.concatenate([fb * hsum, fb], axis=1)
        t12 = jnp.dot(M, kf, preferred_element_type=jnp.float32)
        agg = t12[:, :FEAT] - h * t12[:, FEAT:]
        t = jnp.dot(agg, wo1_ref[b], preferred_element_type=jnp.float32)
        t = _ssp(t + bias_ref[2 + 2 * b:3 + 2 * b, :FEAT])
        out = jnp.dot(t, wo2_ref[b], preferred_element_type=jnp.float32)
        out = out + bias_ref[3 + 2 * b:4 + 2 * b, :FEAT]
        feat = feat + out

    out_ref[0] = feat


def kernel(in_features, embedding_property, emb_table, W_init, W_f1, b_f1,
           W_f2, b_f2, W_o1, b_o1, W_o2, b_o2):
    Fr, B, _ = in_features.shape
    N_EMB, FEAT = emb_table.shape
    G = 64

    pairs = np.asarray(
        [(i, j) for i in range(B) for j in range(i + 1, B)], dtype=np.int32)
    NP_REAL = pairs.shape[0]
    P = -(-NP_REAL // 256) * 256

    M_np = np.zeros((B, P), dtype=np.float32)
    M_np[pairs[:, 0], np.arange(NP_REAL)] = 1.0
    M_np[pairs[:, 1], np.arange(NP_REAL)] = 1.0
    M = jnp.asarray(M_np)
    S = jnp.asarray(M_np.T.copy())

    x = in_features
    ia = np.zeros(P, dtype=np.int32)
    ib = np.zeros(P, dtype=np.int32)
    ia[:NP_REAL] = pairs[:, 0]
    ib[:NP_REAL] = pairs[:, 1]
    XI = jnp.take(x, jnp.asarray(ia), axis=1)
    XJ = jnp.take(x, jnp.asarray(ib), axis=1)
    ep3 = embedding_property.astype(jnp.int32).reshape(Fr, B, 1)
    emb_pad = jnp.pad(emb_table, ((0, 64 - N_EMB), (0, 0)))

    wf1p = jnp.pad(W_f1, ((0, 0), (0, G - _N_GAUSS), (0, 0)))
    W_f1c = jnp.concatenate([wf1p[0], wf1p[1]], axis=1)
    W_f2c = jnp.zeros((2 * FEAT, 2 * FEAT), jnp.float32)
    W_f2c = W_f2c.at[:FEAT, :FEAT].set(W_f2[0]).at[FEAT:, FEAT:].set(W_f2[1])

    def pad128(v):
        return jnp.pad(v, (0, 2 * FEAT - v.shape[0]))

    bias_pack = jnp.stack([
        jnp.concatenate([b_f1[0], b_f1[1]]),
        jnp.concatenate([b_f2[0], b_f2[1]]),
        pad128(b_o1[0]), pad128(b_o2[0]),
        pad128(b_o1[1]), pad128(b_o2[1]),
        jnp.zeros(2 * FEAT), jnp.zeros(2 * FEAT),
    ])

    body = functools.partial(_schnet_body, P=P, B=B, FEAT=FEAT)
    out = pl.pallas_call(
        body,
        grid=(Fr,),
        in_specs=[
            pl.BlockSpec((1, P, 3), lambda f: (f, 0, 0)),
            pl.BlockSpec((1, P, 3), lambda f: (f, 0, 0)),
            pl.BlockSpec((1, B, 1), lambda f: (f, 0, 0)),
            pl.BlockSpec((64, FEAT), lambda f: (0, 0)),
            pl.BlockSpec((2, FEAT, FEAT), lambda f: (0, 0, 0)),
            pl.BlockSpec((G, 2 * FEAT), lambda f: (0, 0)),
            pl.BlockSpec((2 * FEAT, 2 * FEAT), lambda f: (0, 0)),
            pl.BlockSpec((8, 2 * FEAT), lambda f: (0, 0)),
            pl.BlockSpec((2, FEAT, FEAT), lambda f: (0, 0, 0)),
            pl.BlockSpec((2, FEAT, FEAT), lambda f: (0, 0, 0)),
            pl.BlockSpec((B, P), lambda f: (0, 0)),
            pl.BlockSpec((P, B), lambda f: (0, 0)),
        ],
        out_specs=pl.BlockSpec((1, B, FEAT), lambda f: (f, 0, 0)),
        out_shape=jax.ShapeDtypeStruct((Fr, B, FEAT), jnp.float32),
    )(XI, XJ, ep3, emb_pad, W_init, W_f1c, W_f2c, bias_pack, W_o1, W_o2, M, S)
    return out
